# Initial kernel scaffold; baseline (speedup 1.0000x reference)
import jax
import jax.numpy as jnp
from jax.experimental import pallas as pl


def kernel(frag_node_comp1, frag_edge_comp1, frag_edge_index_comp1, frag_batch_comp1, motif_node_comp1, motif_edge_comp1, motif_edge_index_comp1, motif_batch_comp1, frag_node_comp2, frag_edge_comp2, frag_edge_index_comp2, frag_batch_comp2, motif_node_comp2, motif_edge_comp2, motif_edge_index_comp2, motif_batch_comp2, Tb_comp1, Tc_comp1, Tb_comp2, Tc_comp2, params):
    raise NotImplementedError("write your pallas kernel here")


# speedup vs baseline: 1.0002x; 1.0002x over previous
import jax
import jax.numpy as jnp
from jax.experimental import pallas as pl

H = 64; NH = 2; NF = 50000; EF = 800000; NM = 5000; EM = 20000; B = 512


def _linear(p, x):
    return x @ p['w'] + p['b']


def _gru(p, x, h):
    z = jax.nn.sigmoid(x @ p['wz'] + h @ p['uz'] + p['bz'])
    r = jax.nn.sigmoid(x @ p['wr'] + h @ p['ur'] + p['br'])
    n = jnp.tanh(x @ p['wn'] + r * (h @ p['un']) + p['bn'])
    return (1.0 - z) * n + z * h


def _seg_softmax(s, seg, n):
    m = jax.ops.segment_max(s, seg, num_segments=n)
    m = jnp.where(jnp.isfinite(m), m, 0.0)
    e = jnp.exp(s - m[seg])
    d = jax.ops.segment_sum(e, seg, num_segments=n)
    return e / (d[seg] + 1e-9)


def _atom(p, h, ei, ef, n):
    src = ei[0]; dst = ei[1]
    m = jax.nn.leaky_relu(_linear(p['ne'], jnp.concatenate([h[src], ef], axis=-1)))
    s = jax.nn.leaky_relu(_linear(p['align'], jnp.concatenate([h[dst], m], axis=-1)))[:, 0]
    a = _seg_softmax(s, dst, n)
    c = jax.ops.segment_sum(a[:, None] * m, dst, num_segments=n)
    c = jax.nn.elu(_linear(p['attend'], c))
    return _gru(p['gru'], c, h)


def _mol(p, h, seg, n, steps=2):
    sg = jax.ops.segment_sum(h, seg, num_segments=n)
    for _ in range(steps):
        sc = jax.nn.leaky_relu(_linear(p['align'], jnp.concatenate([sg[seg], h], axis=-1)))[:, 0]
        att = _seg_softmax(sc, seg, n)
        c = jax.nn.elu(_linear(p['attend'], jax.ops.segment_sum(att[:, None] * h, seg, num_segments=n)))
        sg = _gru(p['gru'], c, sg)
    return sg


def _component(pc, fn, fe, fei, fb, mn, me, mei, mb):
    x = jax.nn.leaky_relu(_linear(pc['emb_fn'], fn))
    ee = jax.nn.leaky_relu(_linear(pc['emb_fe'], fe))
    outs = []
    for hd in pc['fheads']:
        hh = _atom(hd['atom'], x, fei, ee, NF)
        s = _mol(hd['mol'], hh, fb, NM)
        outs.append(s)
    gm = jax.nn.relu(_linear(pc['frag_attend'], jnp.concatenate(outs, axis=-1)))
    mx = jax.nn.leaky_relu(_linear(pc['emb_mn'], mn))
    mee = jax.nn.leaky_relu(_linear(pc['emb_me'], me))
    mx = jnp.concatenate([gm, mx], axis=-1)
    jo = []
    for hd in pc['jheads']:
        y = _linear(hd['proj'], mx)
        hh = _atom(hd['atom'], y, mei, mee, NM)
        s = _mol(hd['mol'], hh, mb, B)
        jo.append(s)
    return jax.nn.relu(_linear(pc['motif_attend'], jnp.concatenate(jo, axis=-1)))


def _final_mlp_kernel(g1_ref, g2_ref, t_ref, w11, b11, w12, b12, w13, b13,
                      w3, b3, w4, b4, out_ref):
    h1 = jnp.maximum(g1_ref[...] @ w11[...] + b11[...], 0.0)
    h2 = jnp.maximum(g2_ref[...] @ w12[...] + b12[...], 0.0)
    h3 = jnp.maximum(t_ref[...] @ w13[...] + b13[...], 0.0)
    z = jnp.concatenate([h1, h2, h3], axis=-1)
    z = jnp.maximum(z @ w3[...] + b3[...], 0.0)
    out_ref[...] = z @ w4[...] + b4[...]


def kernel(frag_node_comp1, frag_edge_comp1, frag_edge_index_comp1, frag_batch_comp1,
           motif_node_comp1, motif_edge_comp1, motif_edge_index_comp1, motif_batch_comp1,
           frag_node_comp2, frag_edge_comp2, frag_edge_index_comp2, frag_batch_comp2,
           motif_node_comp2, motif_edge_comp2, motif_edge_index_comp2, motif_batch_comp2,
           Tb_comp1, Tc_comp1, Tb_comp2, Tc_comp2, params):
    g1 = _component(params['c1'], frag_node_comp1, frag_edge_comp1, frag_edge_index_comp1,
                    frag_batch_comp1, motif_node_comp1, motif_edge_comp1,
                    motif_edge_index_comp1, motif_batch_comp1)
    g2 = _component(params['c2'], frag_node_comp2, frag_edge_comp2, frag_edge_index_comp2,
                    frag_batch_comp2, motif_node_comp2, motif_edge_comp2,
                    motif_edge_index_comp2, motif_batch_comp2)
    t = jnp.concatenate([Tb_comp1, Tc_comp1, Tb_comp2, Tc_comp2], axis=-1)
    p = params
    return pl.pallas_call(
        _final_mlp_kernel,
        out_shape=jax.ShapeDtypeStruct((B, 1), jnp.float32),
    )(g1, g2, t,
      p['p11']['w'], p['p11']['b'], p['p12']['w'], p['p12']['b'],
      p['p13']['w'], p['p13']['b'], p['p3']['w'], p['p3']['b'],
      p['p4']['w'], p['p4']['b'])


# speedup vs baseline: 3.6451x; 3.6444x over previous
import jax
import jax.numpy as jnp
from jax.experimental import pallas as pl

H = 64; NH = 2; NF = 50000; EF = 800000; NM = 5000; EM = 20000; B = 512


def _gru(p, x, h):
    z = jax.nn.sigmoid(x @ p['wz'] + h @ p['uz'] + p['bz'])
    r = jax.nn.sigmoid(x @ p['wr'] + h @ p['ur'] + p['br'])
    n = jnp.tanh(x @ p['wn'] + r * (h @ p['un']) + p['bn'])
    return (1.0 - z) * n + z * h


def _atom2(heads, xs, ei, ef_emb, n):
    src, dst = ei[0], ei[1]
    W2 = jnp.concatenate([hd['ne']['w'][H:] for hd in heads], axis=1)
    bne = jnp.concatenate([hd['ne']['b'] for hd in heads])
    A2blk = jnp.zeros((2 * H, 2), jnp.float32)
    A2blk = A2blk.at[:H, 0].set(heads[0]['align']['w'][H:, 0])
    A2blk = A2blk.at[H:, 1].set(heads[1]['align']['w'][H:, 0])
    ba = jnp.stack([heads[0]['align']['b'][0], heads[1]['align']['b'][0]])

    G = jnp.concatenate([xs[h] @ heads[h]['ne']['w'][:H] for h in range(2)], axis=1)
    P = jnp.concatenate([xs[h] @ heads[h]['align']['w'][:H] for h in range(2)], axis=1)
    em = ef_emb @ W2 + bne
    m = jax.nn.leaky_relu(G[src] + em)
    s = jax.nn.leaky_relu(P[dst] + m @ A2blk + ba)
    e = jnp.exp(s - jnp.max(s, axis=0))
    num = jax.ops.segment_sum(m * jnp.repeat(e, H, axis=1), dst, num_segments=n)
    den = jax.ops.segment_sum(e, dst, num_segments=n)
    outs = []
    for h, hd in enumerate(heads):
        c = num[:, h * H:(h + 1) * H] / (den[:, h:h + 1] + 1e-9)
        c = jax.nn.elu(c @ hd['attend']['w'] + hd['attend']['b'])
        outs.append(_gru(hd['gru'], c, xs[h]))
    return outs


def _mol(p, h, seg, n, steps=2):
    sg = jax.ops.segment_sum(h, seg, num_segments=n)
    wa1 = p['align']['w'][:H]; wa2 = p['align']['w'][H:]; ba = p['align']['b'][0]
    r = (h @ wa2)[:, 0]
    for _ in range(steps):
        q = (sg @ wa1)[:, 0]
        s = jax.nn.leaky_relu(q[seg] + r + ba)
        e = jnp.exp(s - jnp.max(s))
        num = jax.ops.segment_sum(h * e[:, None], seg, num_segments=n)
        den = jax.ops.segment_sum(e, seg, num_segments=n)
        c = num / (den[:, None] + 1e-9)
        c = jax.nn.elu(c @ p['attend']['w'] + p['attend']['b'])
        sg = _gru(p['gru'], c, sg)
    return sg


def _component(pc, fn, fe, fei, fb, mn, me, mei, mb):
    x = jax.nn.leaky_relu(fn @ pc['emb_fn']['w'] + pc['emb_fn']['b'])
    ee = jax.nn.leaky_relu(fe @ pc['emb_fe']['w'] + pc['emb_fe']['b'])
    hh = _atom2([hd['atom'] for hd in pc['fheads']], [x, x], fei, ee, NF)
    outs = [_mol(pc['fheads'][i]['mol'], hh[i], fb, NM) for i in range(NH)]
    gm = jax.nn.relu(jnp.concatenate(outs, axis=-1) @ pc['frag_attend']['w']
                     + pc['frag_attend']['b'])
    mx = jax.nn.leaky_relu(mn @ pc['emb_mn']['w'] + pc['emb_mn']['b'])
    mee = jax.nn.leaky_relu(me @ pc['emb_me']['w'] + pc['emb_me']['b'])
    mx = jnp.concatenate([gm, mx], axis=-1)
    ys = [mx @ hd['proj']['w'] + hd['proj']['b'] for hd in pc['jheads']]
    hhj = _atom2([hd['atom'] for hd in pc['jheads']], ys, mei, mee, NM)
    jo = [_mol(pc['jheads'][i]['mol'], hhj[i], mb, B) for i in range(NH)]
    return jax.nn.relu(jnp.concatenate(jo, axis=-1) @ pc['motif_attend']['w']
                       + pc['motif_attend']['b'])


def _final_mlp_kernel(g1_ref, g2_ref, t_ref, w11, b11, w12, b12, w13, b13,
                      w3, b3, w4, b4, out_ref):
    h1 = jnp.maximum(g1_ref[...] @ w11[...] + b11[...], 0.0)
    h2 = jnp.maximum(g2_ref[...] @ w12[...] + b12[...], 0.0)
    h3 = jnp.maximum(t_ref[...] @ w13[...] + b13[...], 0.0)
    z = jnp.concatenate([h1, h2, h3], axis=-1)
    z = jnp.maximum(z @ w3[...] + b3[...], 0.0)
    out_ref[...] = z @ w4[...] + b4[...]


def kernel(frag_node_comp1, frag_edge_comp1, frag_edge_index_comp1, frag_batch_comp1,
           motif_node_comp1, motif_edge_comp1, motif_edge_index_comp1, motif_batch_comp1,
           frag_node_comp2, frag_edge_comp2, frag_edge_index_comp2, frag_batch_comp2,
           motif_node_comp2, motif_edge_comp2, motif_edge_index_comp2, motif_batch_comp2,
           Tb_comp1, Tc_comp1, Tb_comp2, Tc_comp2, params):
    g1 = _component(params['c1'], frag_node_comp1, frag_edge_comp1, frag_edge_index_comp1,
                    frag_batch_comp1, motif_node_comp1, motif_edge_comp1,
                    motif_edge_index_comp1, motif_batch_comp1)
    g2 = _component(params['c2'], frag_node_comp2, frag_edge_comp2, frag_edge_index_comp2,
                    frag_batch_comp2, motif_node_comp2, motif_edge_comp2,
                    motif_edge_index_comp2, motif_batch_comp2)
    t = jnp.concatenate([Tb_comp1, Tc_comp1, Tb_comp2, Tc_comp2], axis=-1)
    p = params
    return pl.pallas_call(
        _final_mlp_kernel,
        out_shape=jax.ShapeDtypeStruct((B, 1), jnp.float32),
    )(g1, g2, t,
      p['p11']['w'], p['p11']['b'], p['p12']['w'], p['p12']['b'],
      p['p13']['w'], p['p13']['b'], p['p3']['w'], p['p3']['b'],
      p['p4']['w'], p['p4']['b'])


# speedup vs baseline: 4.4694x; 1.2262x over previous
import functools

import jax
import jax.numpy as jnp
from jax import lax
from jax.experimental import pallas as pl
from jax.experimental.pallas import tpu as pltpu
from jax.experimental.pallas import tpu_sc as plsc

H = 64; NH = 2; NF = 50000; EF = 800000; NM = 5000; EM = 20000; B = 512
NW = 32
CH = 128


def _ceil_to(x, m):
    return (x + m - 1) // m * m


_SC_SCATTER_ENABLED = True


def _sc_gather(table, idx, tc_tiling=True):
    N, D = table.shape
    E = idx.shape[0]
    per_w = E // NW
    nch = per_w // CH
    mesh = plsc.VectorSubcoreMesh(core_axis_name="c", subcore_axis_name="s")

    @functools.partial(
        pl.kernel, mesh=mesh,
        out_type=jax.ShapeDtypeStruct((E, D), jnp.float32),
        compiler_params=pltpu.CompilerParams(use_tc_tiling_on_sc=tc_tiling),
        scratch_types=[pltpu.VMEM((CH,), jnp.int32),
                       pltpu.VMEM((CH, D), jnp.float32),
                       pltpu.SemaphoreType.DMA])
    def k(table_hbm, idx_hbm, out_hbm, idx_v, rows_v, sem):
        wid = lax.axis_index("s") * 2 + lax.axis_index("c")
        base = wid * per_w

        def body(g, carry):
            off = base + g * CH
            pltpu.sync_copy(idx_hbm.at[pl.ds(off, CH)], idx_v)
            pltpu.async_copy(table_hbm.at[idx_v], rows_v, sem).wait()
            pltpu.sync_copy(rows_v, out_hbm.at[pl.ds(off, CH)])
            return carry

        lax.fori_loop(0, nch, body, 0)

    return k(table, idx)


def _sc_scatter(pay, idx, nseg):
    R, E, D = pay.shape
    half = E // 2
    per_t = half // 16
    nch = per_t // CH
    rows_pt = nseg // 16
    zeros = jnp.zeros((nseg, D), jnp.float32)
    mesh = plsc.VectorSubcoreMesh(core_axis_name="c", subcore_axis_name="s")

    @functools.partial(
        pl.kernel, mesh=mesh,
        out_type=jax.ShapeDtypeStruct((R, 2, nseg, D), jnp.float32),
        compiler_params=pltpu.CompilerParams(use_tc_tiling_on_sc=False),
        scratch_types=[pltpu.VMEM((CH,), jnp.int32),
                       pltpu.VMEM((CH, D), jnp.float32),
                       pltpu.VMEM_SHARED((nseg, D), jnp.float32)])
    def k(pay_hbm, idx_hbm, zero_hbm, out_hbm, idx_v, pay_v, acc):
        c = lax.axis_index("c")
        s = lax.axis_index("s")
        for r in range(R):
            pltpu.sync_copy(zero_hbm.at[pl.ds(s * rows_pt, rows_pt)],
                            acc.at[pl.ds(s * rows_pt, rows_pt)])
            plsc.subcore_barrier()

            def body(g, carry):
                off = c * half + s * per_t + g * CH
                pltpu.sync_copy(idx_hbm.at[pl.ds(off, CH)], idx_v)
                pltpu.sync_copy(pay_hbm.at[r, pl.ds(off, CH)], pay_v)
                pltpu.sync_copy(pay_v, acc.at[idx_v], add=True)
                return carry

            lax.fori_loop(0, nch, body, 0)
            plsc.subcore_barrier()
            pltpu.sync_copy(acc.at[pl.ds(s * rows_pt, rows_pt)],
                            out_hbm.at[r, c, pl.ds(s * rows_pt, rows_pt)])
            plsc.subcore_barrier()

    return k(pay, idx, zeros)


def _leaky(x):
    return jnp.where(x >= 0, x, 0.01 * x)


def _node_a_body(fn_ref, wfn, bfn, w1, a1, x_ref, g_ref, p_ref):
    x = _leaky(fn_ref[...] @ wfn[...] + bfn[...])
    x_ref[...] = x
    g_ref[...] = x @ w1[...]
    p_ref[...] = jnp.pad(x @ a1[...], ((0, 0), (0, 14)))


def _tc_node_a(fn, wfn, bfn, w1cat, a1cat, blk=2000):
    n = fn.shape[0]
    grid = (n // blk,)
    return pl.pallas_call(
        _node_a_body,
        grid=grid,
        in_specs=[pl.BlockSpec((blk, fn.shape[1]), lambda i: (i, 0)),
                  pl.BlockSpec(wfn.shape, lambda i: (0, 0)),
                  pl.BlockSpec(bfn.shape, lambda i: (0, 0)),
                  pl.BlockSpec(w1cat.shape, lambda i: (0, 0)),
                  pl.BlockSpec(a1cat.shape, lambda i: (0, 0))],
        out_specs=[pl.BlockSpec((blk, H), lambda i: (i, 0)),
                   pl.BlockSpec((blk, 2 * H), lambda i: (i, 0)),
                   pl.BlockSpec((blk, 16), lambda i: (i, 0))],
        out_shape=[jax.ShapeDtypeStruct((n, H), jnp.float32),
                   jax.ShapeDtypeStruct((n, 2 * H), jnp.float32),
                   jax.ShapeDtypeStruct((n, 16), jnp.float32)],
    )(fn, wfn, bfn, w1cat, a1cat)


def _edge_m(gs, fe, we, be, w2, bne):
    ee = _leaky(fe @ we + be)
    return _leaky(gs + ee @ w2 + bne)


def _pass_a_body(gs_ref, fe_ref, pd_ref, we, be, w2, bne, a2, ba,
                 s_ref, m_ref, *, blocks_per_comp):
    i = pl.program_id(0)
    m = _edge_m(gs_ref[...], fe_ref[...], we[0], be[0], w2[0], bne[0])
    s = _leaky(pd_ref[...][:, :2] + m @ a2[0] + ba[0])
    s_ref[...] = jnp.pad(s, ((0, 0), (0, 6)))

    @pl.when(i == 0)
    def _():
        m_ref[...] = jnp.full((2, 8), -1e30, jnp.float32)

    comp = i // blocks_per_comp
    rows = lax.broadcasted_iota(jnp.int32, (2, 8), 0)
    smax = jnp.pad(jnp.max(s, axis=0)[None, :], ((0, 0), (0, 6)),
                   constant_values=-1e30)
    upd = jnp.where(rows == comp, jnp.broadcast_to(smax, (2, 8)), -1e30)
    m_ref[...] = jnp.maximum(m_ref[...], upd)


def _tc_pass_a(gs, fe, pd, we, be, w2, bne, a2, ba, blk=4096):
    ep = gs.shape[0]
    bpc = ep // 2 // blk
    grid = (ep // blk,)
    wspec = lambda a: pl.BlockSpec(
        (1,) + a.shape[1:], lambda i: (i // bpc,) + tuple(0 for _ in a.shape[1:]))
    return pl.pallas_call(
        functools.partial(_pass_a_body, blocks_per_comp=bpc),
        grid=grid,
        in_specs=[pl.BlockSpec((blk, 2 * H), lambda i: (i, 0)),
                  pl.BlockSpec((blk, fe.shape[1]), lambda i: (i, 0)),
                  pl.BlockSpec((blk, 16), lambda i: (i, 0)),
                  wspec(we), wspec(be), wspec(w2), wspec(bne), wspec(a2),
                  wspec(ba)],
        out_specs=[pl.BlockSpec((blk, 8), lambda i: (i, 0)),
                   pl.BlockSpec((2, 8), lambda i: (0, 0))],
        out_shape=[jax.ShapeDtypeStruct((ep, 8), jnp.float32),
                   jax.ShapeDtypeStruct((2, 8), jnp.float32)],
    )(gs, fe, pd, we, be, w2, bne, a2, ba)


def _pass_b_body(gs_ref, fe_ref, s_ref, mx_ref, we, be, w2, bne, pay_ref, *,
                 blocks_per_comp):
    i = pl.program_id(0)
    comp = i // blocks_per_comp
    m = _edge_m(gs_ref[...], fe_ref[...], we[0], be[0], w2[0], bne[0])
    rows = lax.broadcasted_iota(jnp.int32, (2, 8), 0)
    mrow = jnp.sum(jnp.where(rows == comp, mx_ref[...], 0.0), axis=0)
    e = jnp.exp(s_ref[...][:, :2] - mrow[None, :2])
    e0 = e[:, :1]
    e1 = e[:, 1:2]
    pay_ref[0] = jnp.pad(e, ((0, 0), (0, 30)))
    pay_ref[1] = m[:, 0:32] * e0
    pay_ref[2] = m[:, 32:64] * e0
    pay_ref[3] = m[:, 64:96] * e1
    pay_ref[4] = m[:, 96:128] * e1


def _tc_pass_b(gs, fe, s, mx, we, be, w2, bne, blk=4096):
    ep = gs.shape[0]
    bpc = ep // 2 // blk
    grid = (ep // blk,)
    wspec = lambda a: pl.BlockSpec(
        (1,) + a.shape[1:], lambda i: (i // bpc,) + tuple(0 for _ in a.shape[1:]))
    return pl.pallas_call(
        functools.partial(_pass_b_body, blocks_per_comp=bpc),
        grid=grid,
        in_specs=[pl.BlockSpec((blk, 2 * H), lambda i: (i, 0)),
                  pl.BlockSpec((blk, fe.shape[1]), lambda i: (i, 0)),
                  pl.BlockSpec((blk, 8), lambda i: (i, 0)),
                  pl.BlockSpec((2, 8), lambda i: (0, 0)),
                  wspec(we), wspec(be), wspec(w2), wspec(bne)],
        out_specs=[pl.BlockSpec((5, blk, 32), lambda i: (0, i, 0))],
        out_shape=[jax.ShapeDtypeStruct((5, ep, 32), jnp.float32)],
    )(gs, fe, s, mx, we, be, w2, bne)[0]


def _node_b_body(scat_ref, x_ref, wat, bat, wg, ug, bg, hh_ref):
    for h in range(NH):
        num = jnp.concatenate([scat_ref[1 + 2 * h, 0], scat_ref[2 + 2 * h, 0]],
                              axis=1)
        den = scat_ref[0, 0][:, h:h + 1]
        cc = num / (den + 1e-9)
        cc = cc @ wat[0, h] + bat[0, h]
        cc = jnp.where(cc > 0, cc, jnp.exp(jnp.minimum(cc, 0.0)) - 1.0)
        hx = x_ref[0]
        z = jax.nn.sigmoid(cc @ wg[0, h, 0] + hx @ ug[0, h, 0] + bg[0, h, 0])
        r = jax.nn.sigmoid(cc @ wg[0, h, 1] + hx @ ug[0, h, 1] + bg[0, h, 1])
        nn = jnp.tanh(cc @ wg[0, h, 2] + r * (hx @ ug[0, h, 2]) + bg[0, h, 2])
        hh_ref[0, h] = (1.0 - z) * nn + z * hx


def _tc_node_b(scat, xs, wat, bat, wg, ug, bg, n, blk=2000):
    nseg = scat.shape[2]
    grid = (2, n // blk)
    full = lambda a: pl.BlockSpec(
        (1,) + a.shape[1:], lambda c, i: (c,) + tuple(0 for _ in a.shape[1:]))
    return pl.pallas_call(
        _node_b_body,
        grid=grid,
        in_specs=[pl.BlockSpec((5, 1, blk, 32), lambda c, i: (0, c, i, 0)),
                  pl.BlockSpec((1, blk, H), lambda c, i: (c, i, 0)),
                  full(wat), full(bat), full(wg), full(ug), full(bg)],
        out_specs=[pl.BlockSpec((1, NH, blk, H), lambda c, i: (c, 0, i, 0))],
        out_shape=[jax.ShapeDtypeStruct((2, NH, n, H), jnp.float32)],
    )(scat, xs, wat, bat, wg, ug, bg)[0]


def _frag_stage(pc1, pc2, fn1, fe1, ei1, fn2, fe2, ei2):
    epc = _ceil_to(EF, 4096)
    ep = 2 * epc
    nseg = _ceil_to(NF + 1, 128)

    def prep(pc):
        heads = [hd['atom'] for hd in pc['fheads']]
        w1 = jnp.concatenate([hd['ne']['w'][:H] for hd in heads], axis=1)
        w2 = jnp.concatenate([hd['ne']['w'][H:] for hd in heads], axis=1)
        bne = jnp.concatenate([hd['ne']['b'] for hd in heads])[None, :]
        a1 = jnp.concatenate([hd['align']['w'][:H] for hd in heads], axis=1)
        a2 = jnp.zeros((2 * H, 2), jnp.float32)
        a2 = a2.at[:H, 0].set(heads[0]['align']['w'][H:, 0])
        a2 = a2.at[H:, 1].set(heads[1]['align']['w'][H:, 0])
        ba = jnp.stack([heads[0]['align']['b'][0],
                        heads[1]['align']['b'][0]])[None, :]
        wat = jnp.stack([hd['attend']['w'] for hd in heads])
        bat = jnp.stack([hd['attend']['b'] for hd in heads])[:, None, :]
        wg = jnp.stack([jnp.stack([hd['gru']['wz'], hd['gru']['wr'],
                                   hd['gru']['wn']]) for hd in heads])
        ug = jnp.stack([jnp.stack([hd['gru']['uz'], hd['gru']['ur'],
                                   hd['gru']['un']]) for hd in heads])
        bg = jnp.stack([jnp.stack([hd['gru']['bz'], hd['gru']['br'],
                                   hd['gru']['bn']]) for hd in heads])[:, :, None, :]
        return dict(w1=w1, w2=w2, bne=bne, a1=a1, a2=a2, ba=ba, wat=wat,
                    bat=bat, wg=wg, ug=ug, bg=bg)

    p1, p2 = prep(pc1), prep(pc2)

    x1, g1, pt1 = _tc_node_a(fn1, pc1['emb_fn']['w'], pc1['emb_fn']['b'][None, :],
                             p1['w1'], p1['a1'])
    x2, g2, pt2 = _tc_node_a(fn2, pc2['emb_fn']['w'], pc2['emb_fn']['b'][None, :],
                             p2['w1'], p2['a1'])

    gcat = jnp.concatenate([g1, g2])
    ptcat = jnp.concatenate([pt1, pt2])

    def padi(a, n, v):
        return jnp.pad(a, (0, n - a.shape[0]), constant_values=v)

    src_g = jnp.concatenate([padi(ei1[0], epc, 0), padi(ei2[0] + NF, epc, NF)])
    dst_g = jnp.concatenate([padi(ei1[1], epc, 0), padi(ei2[1] + NF, epc, NF)])
    dst_s = jnp.concatenate([padi(ei1[1], epc, NF), padi(ei2[1], epc, NF)])

    gs = _sc_gather(gcat, src_g)
    pd = _sc_gather(ptcat, dst_g, tc_tiling=False)

    def pade(a, n):
        return jnp.pad(a, ((0, n - a.shape[0]), (0, 0)))

    fecat = jnp.concatenate([pade(fe1, epc), pade(fe2, epc)])
    we = jnp.stack([pc1['emb_fe']['w'], pc2['emb_fe']['w']])
    be = jnp.stack([pc1['emb_fe']['b'][None, :], pc2['emb_fe']['b'][None, :]])
    w2 = jnp.stack([p1['w2'], p2['w2']])
    bne = jnp.stack([p1['bne'], p2['bne']])
    a2 = jnp.stack([p1['a2'], p2['a2']])
    ba = jnp.stack([p1['ba'], p2['ba']])
    s, mx = _tc_pass_a(gs, fecat, pd, we, be, w2, bne, a2, ba)
    pay = _tc_pass_b(gs, fecat, s, mx, we, be, w2, bne)
    if _SC_SCATTER_ENABLED:
        scat = _sc_scatter(pay, dst_s, nseg)
    else:
        seg = jnp.where(jnp.arange(ep) < epc, dst_s, dst_s + nseg)
        sc2 = jax.vmap(lambda p: jax.ops.segment_sum(p, seg, num_segments=2 * nseg))(pay)
        scat = sc2.reshape(5, 2, nseg, 32)
    scat = scat[:, :, :NF, :]

    xs = jnp.stack([x1, x2])
    wat = jnp.stack([p1['wat'], p2['wat']])
    bat = jnp.stack([p1['bat'], p2['bat']])
    wg = jnp.stack([p1['wg'], p2['wg']])
    ug = jnp.stack([p1['ug'], p2['ug']])
    bg = jnp.stack([p1['bg'], p2['bg']])
    return _tc_node_b(scat, xs, wat, bat, wg, ug, bg, NF)


def _gru(p, x, h):
    z = jax.nn.sigmoid(x @ p['wz'] + h @ p['uz'] + p['bz'])
    r = jax.nn.sigmoid(x @ p['wr'] + h @ p['ur'] + p['br'])
    n = jnp.tanh(x @ p['wn'] + r * (h @ p['un']) + p['bn'])
    return (1.0 - z) * n + z * h


def _atom2(heads, xs, ei, ef_emb, n):
    src, dst = ei[0], ei[1]
    W2 = jnp.concatenate([hd['ne']['w'][H:] for hd in heads], axis=1)
    bne = jnp.concatenate([hd['ne']['b'] for hd in heads])
    A2blk = jnp.zeros((2 * H, 2), jnp.float32)
    A2blk = A2blk.at[:H, 0].set(heads[0]['align']['w'][H:, 0])
    A2blk = A2blk.at[H:, 1].set(heads[1]['align']['w'][H:, 0])
    ba = jnp.stack([heads[0]['align']['b'][0], heads[1]['align']['b'][0]])
    G = jnp.concatenate([xs[h] @ heads[h]['ne']['w'][:H] for h in range(2)], axis=1)
    P = jnp.concatenate([xs[h] @ heads[h]['align']['w'][:H] for h in range(2)], axis=1)
    em = ef_emb @ W2 + bne
    m = jax.nn.leaky_relu(G[src] + em)
    s = jax.nn.leaky_relu(P[dst] + m @ A2blk + ba)
    e = jnp.exp(s - jnp.max(s, axis=0))
    num = jax.ops.segment_sum(m * jnp.repeat(e, H, axis=1), dst, num_segments=n)
    den = jax.ops.segment_sum(e, dst, num_segments=n)
    outs = []
    for h, hd in enumerate(heads):
        c = num[:, h * H:(h + 1) * H] / (den[:, h:h + 1] + 1e-9)
        c = jax.nn.elu(c @ hd['attend']['w'] + hd['attend']['b'])
        outs.append(_gru(hd['gru'], c, xs[h]))
    return outs


def _mol(p, h, seg, n, steps=2):
    sg = jax.ops.segment_sum(h, seg, num_segments=n)
    wa1 = p['align']['w'][:H]; wa2 = p['align']['w'][H:]; ba = p['align']['b'][0]
    r = (h @ wa2)[:, 0]
    for _ in range(steps):
        q = (sg @ wa1)[:, 0]
        s = jax.nn.leaky_relu(q[seg] + r + ba)
        e = jnp.exp(s - jnp.max(s))
        num = jax.ops.segment_sum(h * e[:, None], seg, num_segments=n)
        den = jax.ops.segment_sum(e, seg, num_segments=n)
        c = num / (den[:, None] + 1e-9)
        c = jax.nn.elu(c @ p['attend']['w'] + p['attend']['b'])
        sg = _gru(p['gru'], c, sg)
    return sg


def _component_tail(pc, hh, fb, mn, me, mei, mb):
    outs = [_mol(pc['fheads'][i]['mol'], hh[i], fb, NM) for i in range(NH)]
    gm = jax.nn.relu(jnp.concatenate(outs, axis=-1) @ pc['frag_attend']['w']
                     + pc['frag_attend']['b'])
    mx = jax.nn.leaky_relu(mn @ pc['emb_mn']['w'] + pc['emb_mn']['b'])
    mee = jax.nn.leaky_relu(me @ pc['emb_me']['w'] + pc['emb_me']['b'])
    mx = jnp.concatenate([gm, mx], axis=-1)
    ys = [mx @ hd['proj']['w'] + hd['proj']['b'] for hd in pc['jheads']]
    hhj = _atom2([hd['atom'] for hd in pc['jheads']], ys, mei, mee, NM)
    jo = [_mol(pc['jheads'][i]['mol'], hhj[i], mb, B) for i in range(NH)]
    return jax.nn.relu(jnp.concatenate(jo, axis=-1) @ pc['motif_attend']['w']
                       + pc['motif_attend']['b'])


def _final_mlp_kernel(g1_ref, g2_ref, t_ref, w11, b11, w12, b12, w13, b13,
                      w3, b3, w4, b4, out_ref):
    h1 = jnp.maximum(g1_ref[...] @ w11[...] + b11[...], 0.0)
    h2 = jnp.maximum(g2_ref[...] @ w12[...] + b12[...], 0.0)
    h3 = jnp.maximum(t_ref[...] @ w13[...] + b13[...], 0.0)
    z = jnp.concatenate([h1, h2, h3], axis=-1)
    z = jnp.maximum(z @ w3[...] + b3[...], 0.0)
    out_ref[...] = z @ w4[...] + b4[...]


def kernel(frag_node_comp1, frag_edge_comp1, frag_edge_index_comp1, frag_batch_comp1,
           motif_node_comp1, motif_edge_comp1, motif_edge_index_comp1, motif_batch_comp1,
           frag_node_comp2, frag_edge_comp2, frag_edge_index_comp2, frag_batch_comp2,
           motif_node_comp2, motif_edge_comp2, motif_edge_index_comp2, motif_batch_comp2,
           Tb_comp1, Tc_comp1, Tb_comp2, Tc_comp2, params):
    pc1, pc2 = params['c1'], params['c2']
    hh = _frag_stage(pc1, pc2, frag_node_comp1, frag_edge_comp1,
                     frag_edge_index_comp1, frag_node_comp2, frag_edge_comp2,
                     frag_edge_index_comp2)
    g1 = _component_tail(pc1, [hh[0, 0], hh[0, 1]], frag_batch_comp1,
                         motif_node_comp1, motif_edge_comp1,
                         motif_edge_index_comp1, motif_batch_comp1)
    g2 = _component_tail(pc2, [hh[1, 0], hh[1, 1]], frag_batch_comp2,
                         motif_node_comp2, motif_edge_comp2,
                         motif_edge_index_comp2, motif_batch_comp2)
    t = jnp.concatenate([Tb_comp1, Tc_comp1, Tb_comp2, Tc_comp2], axis=-1)
    p = params
    return pl.pallas_call(
        _final_mlp_kernel,
        out_shape=jax.ShapeDtypeStruct((B, 1), jnp.float32),
    )(g1, g2, t,
      p['p11']['w'], p['p11']['b'], p['p12']['w'], p['p12']['b'],
      p['p13']['w'], p['p13']['b'], p['p3']['w'], p['p3']['b'],
      p['p4']['w'], p['p4']['b'])


# speedup vs baseline: 5.0802x; 1.1367x over previous
import functools

import jax
import jax.numpy as jnp
from jax import lax
from jax.experimental import pallas as pl
from jax.experimental.pallas import tpu as pltpu
from jax.experimental.pallas import tpu_sc as plsc

H = 64; NH = 2; NF = 50000; EF = 800000; NM = 5000; EM = 20000; B = 512
NW = 32
CH = 128


def _ceil_to(x, m):
    return (x + m - 1) // m * m


_SC_SCATTER_ENABLED = True


def _sc_gather(table, idx, tc_tiling=True):
    N, D = table.shape
    E = idx.shape[0]
    per_w = E // NW
    nch = per_w // CH
    mesh = plsc.VectorSubcoreMesh(core_axis_name="c", subcore_axis_name="s")

    @functools.partial(
        pl.kernel, mesh=mesh,
        out_type=jax.ShapeDtypeStruct((E, D), jnp.float32),
        compiler_params=pltpu.CompilerParams(use_tc_tiling_on_sc=tc_tiling),
        scratch_types=[pltpu.VMEM((CH,), jnp.int32),
                       pltpu.VMEM((CH, D), jnp.float32),
                       pltpu.SemaphoreType.DMA])
    def k(table_hbm, idx_hbm, out_hbm, idx_v, rows_v, sem):
        wid = lax.axis_index("s") * 2 + lax.axis_index("c")
        base = wid * per_w

        def body(g, carry):
            off = base + g * CH
            pltpu.sync_copy(idx_hbm.at[pl.ds(off, CH)], idx_v)
            pltpu.async_copy(table_hbm.at[idx_v], rows_v, sem).wait()
            pltpu.sync_copy(rows_v, out_hbm.at[pl.ds(off, CH)])
            return carry

        lax.fori_loop(0, nch, body, 0)

    return k(table, idx)


def _sc_scatter(pay, idx, nseg):
    R, E, D = pay.shape
    half = E // 2
    per_t = half // 16
    nch = per_t // CH
    rows_pt = nseg // 16
    zeros = jnp.zeros((nseg, D), jnp.float32)
    mesh = plsc.VectorSubcoreMesh(core_axis_name="c", subcore_axis_name="s")

    @functools.partial(
        pl.kernel, mesh=mesh,
        out_type=jax.ShapeDtypeStruct((R, 2, nseg, D), jnp.float32),
        compiler_params=pltpu.CompilerParams(use_tc_tiling_on_sc=False),
        scratch_types=[pltpu.VMEM((CH,), jnp.int32),
                       pltpu.VMEM((CH, D), jnp.float32),
                       pltpu.VMEM_SHARED((nseg, D), jnp.float32)])
    def k(pay_hbm, idx_hbm, zero_hbm, out_hbm, idx_v, pay_v, acc):
        c = lax.axis_index("c")
        s = lax.axis_index("s")
        for r in range(R):
            pltpu.sync_copy(zero_hbm.at[pl.ds(s * rows_pt, rows_pt)],
                            acc.at[pl.ds(s * rows_pt, rows_pt)])
            plsc.subcore_barrier()

            def body(g, carry):
                off = c * half + s * per_t + g * CH
                pltpu.sync_copy(idx_hbm.at[pl.ds(off, CH)], idx_v)
                pltpu.sync_copy(pay_hbm.at[r, pl.ds(off, CH)], pay_v)
                pltpu.sync_copy(pay_v, acc.at[idx_v], add=True)
                return carry

            lax.fori_loop(0, nch, body, 0)
            plsc.subcore_barrier()
            pltpu.sync_copy(acc.at[pl.ds(s * rows_pt, rows_pt)],
                            out_hbm.at[r, c, pl.ds(s * rows_pt, rows_pt)])
            plsc.subcore_barrier()

    return k(pay, idx, zeros)


def _leaky(x):
    return jnp.where(x >= 0, x, 0.01 * x)


def _node_a_body(fn_ref, wfn, bfn, w1, a1, x_ref, g_ref, p_ref):
    x = _leaky(fn_ref[...] @ wfn[...] + bfn[...])
    x_ref[...] = x
    g_ref[...] = x @ w1[...]
    p_ref[...] = jnp.pad(x @ a1[...], ((0, 0), (0, 14)))


def _tc_node_a(fn, wfn, bfn, w1cat, a1cat, blk=2000):
    n = fn.shape[0]
    grid = (n // blk,)
    return pl.pallas_call(
        _node_a_body,
        grid=grid,
        in_specs=[pl.BlockSpec((blk, fn.shape[1]), lambda i: (i, 0)),
                  pl.BlockSpec(wfn.shape, lambda i: (0, 0)),
                  pl.BlockSpec(bfn.shape, lambda i: (0, 0)),
                  pl.BlockSpec(w1cat.shape, lambda i: (0, 0)),
                  pl.BlockSpec(a1cat.shape, lambda i: (0, 0))],
        out_specs=[pl.BlockSpec((blk, H), lambda i: (i, 0)),
                   pl.BlockSpec((blk, 2 * H), lambda i: (i, 0)),
                   pl.BlockSpec((blk, 16), lambda i: (i, 0))],
        out_shape=[jax.ShapeDtypeStruct((n, H), jnp.float32),
                   jax.ShapeDtypeStruct((n, 2 * H), jnp.float32),
                   jax.ShapeDtypeStruct((n, 16), jnp.float32)],
    )(fn, wfn, bfn, w1cat, a1cat)


def _edge_m(gs, fe, we, be, w2, bne):
    ee = _leaky(fe @ we + be)
    return _leaky(gs + ee @ w2 + bne)


def _pass_a_body(gs_ref, fe_ref, pd_ref, we, be, w2, bne, a2, ba,
                 s_ref, m_ref, *, blocks_per_comp):
    i = pl.program_id(0)
    m = _edge_m(gs_ref[...], fe_ref[...], we[0], be[0], w2[0], bne[0])
    s = _leaky(pd_ref[...][:, :2] + m @ a2[0] + ba[0])
    s_ref[...] = jnp.pad(s, ((0, 0), (0, 6)))

    @pl.when(i == 0)
    def _():
        m_ref[...] = jnp.full((2, 8), -1e30, jnp.float32)

    comp = i // blocks_per_comp
    rows = lax.broadcasted_iota(jnp.int32, (2, 8), 0)
    smax = jnp.pad(jnp.max(s, axis=0)[None, :], ((0, 0), (0, 6)),
                   constant_values=-1e30)
    upd = jnp.where(rows == comp, jnp.broadcast_to(smax, (2, 8)), -1e30)
    m_ref[...] = jnp.maximum(m_ref[...], upd)


def _tc_pass_a(gs, fe, pd, we, be, w2, bne, a2, ba, blk=4096):
    ep = gs.shape[0]
    bpc = ep // 2 // blk
    grid = (ep // blk,)
    wspec = lambda a: pl.BlockSpec(
        (1,) + a.shape[1:], lambda i: (i // bpc,) + tuple(0 for _ in a.shape[1:]))
    return pl.pallas_call(
        functools.partial(_pass_a_body, blocks_per_comp=bpc),
        grid=grid,
        in_specs=[pl.BlockSpec((blk, 2 * H), lambda i: (i, 0)),
                  pl.BlockSpec((blk, fe.shape[1]), lambda i: (i, 0)),
                  pl.BlockSpec((blk, 16), lambda i: (i, 0)),
                  wspec(we), wspec(be), wspec(w2), wspec(bne), wspec(a2),
                  wspec(ba)],
        out_specs=[pl.BlockSpec((blk, 8), lambda i: (i, 0)),
                   pl.BlockSpec((2, 8), lambda i: (0, 0))],
        out_shape=[jax.ShapeDtypeStruct((ep, 8), jnp.float32),
                   jax.ShapeDtypeStruct((2, 8), jnp.float32)],
    )(gs, fe, pd, we, be, w2, bne, a2, ba)


def _pass_b_body(gs_ref, fe_ref, s_ref, mx_ref, we, be, w2, bne, pay_ref, *,
                 blocks_per_comp):
    i = pl.program_id(0)
    comp = i // blocks_per_comp
    m = _edge_m(gs_ref[...], fe_ref[...], we[0], be[0], w2[0], bne[0])
    rows = lax.broadcasted_iota(jnp.int32, (2, 8), 0)
    mrow = jnp.sum(jnp.where(rows == comp, mx_ref[...], 0.0), axis=0)
    e = jnp.exp(s_ref[...][:, :2] - mrow[None, :2])
    e0 = e[:, :1]
    e1 = e[:, 1:2]
    pay_ref[0] = jnp.pad(e, ((0, 0), (0, 30)))
    pay_ref[1] = m[:, 0:32] * e0
    pay_ref[2] = m[:, 32:64] * e0
    pay_ref[3] = m[:, 64:96] * e1
    pay_ref[4] = m[:, 96:128] * e1


def _tc_pass_b(gs, fe, s, mx, we, be, w2, bne, blk=4096):
    ep = gs.shape[0]
    bpc = ep // 2 // blk
    grid = (ep // blk,)
    wspec = lambda a: pl.BlockSpec(
        (1,) + a.shape[1:], lambda i: (i // bpc,) + tuple(0 for _ in a.shape[1:]))
    return pl.pallas_call(
        functools.partial(_pass_b_body, blocks_per_comp=bpc),
        grid=grid,
        in_specs=[pl.BlockSpec((blk, 2 * H), lambda i: (i, 0)),
                  pl.BlockSpec((blk, fe.shape[1]), lambda i: (i, 0)),
                  pl.BlockSpec((blk, 8), lambda i: (i, 0)),
                  pl.BlockSpec((2, 8), lambda i: (0, 0)),
                  wspec(we), wspec(be), wspec(w2), wspec(bne)],
        out_specs=[pl.BlockSpec((5, blk, 32), lambda i: (0, i, 0))],
        out_shape=[jax.ShapeDtypeStruct((5, ep, 32), jnp.float32)],
    )(gs, fe, s, mx, we, be, w2, bne)[0]


def _node_b_body(scat_ref, x_ref, wat, bat, wg, ug, bg, hh_ref):
    for h in range(NH):
        num = jnp.concatenate([scat_ref[1 + 2 * h, 0], scat_ref[2 + 2 * h, 0]],
                              axis=1)
        den = scat_ref[0, 0][:, h:h + 1]
        cc = num / (den + 1e-9)
        cc = cc @ wat[0, h] + bat[0, h]
        cc = jnp.where(cc > 0, cc, jnp.exp(jnp.minimum(cc, 0.0)) - 1.0)
        hx = x_ref[0]
        z = jax.nn.sigmoid(cc @ wg[0, h, 0] + hx @ ug[0, h, 0] + bg[0, h, 0])
        r = jax.nn.sigmoid(cc @ wg[0, h, 1] + hx @ ug[0, h, 1] + bg[0, h, 1])
        nn = jnp.tanh(cc @ wg[0, h, 2] + r * (hx @ ug[0, h, 2]) + bg[0, h, 2])
        hh_ref[0, h] = (1.0 - z) * nn + z * hx


def _tc_node_b(scat, xs, wat, bat, wg, ug, bg, n, blk=2000):
    nseg = scat.shape[2]
    grid = (2, n // blk)
    full = lambda a: pl.BlockSpec(
        (1,) + a.shape[1:], lambda c, i: (c,) + tuple(0 for _ in a.shape[1:]))
    return pl.pallas_call(
        _node_b_body,
        grid=grid,
        in_specs=[pl.BlockSpec((5, 1, blk, 32), lambda c, i: (0, c, i, 0)),
                  pl.BlockSpec((1, blk, H), lambda c, i: (c, i, 0)),
                  full(wat), full(bat), full(wg), full(ug), full(bg)],
        out_specs=[pl.BlockSpec((1, NH, blk, H), lambda c, i: (c, 0, i, 0))],
        out_shape=[jax.ShapeDtypeStruct((2, NH, n, H), jnp.float32)],
    )(scat, xs, wat, bat, wg, ug, bg)[0]


def _frag_stage(pc1, pc2, fn1, fe1, ei1, fn2, fe2, ei2):
    epc = _ceil_to(EF, 4096)
    ep = 2 * epc
    nseg = _ceil_to(NF + 1, 128)

    def prep(pc):
        heads = [hd['atom'] for hd in pc['fheads']]
        w1 = jnp.concatenate([hd['ne']['w'][:H] for hd in heads], axis=1)
        w2 = jnp.concatenate([hd['ne']['w'][H:] for hd in heads], axis=1)
        bne = jnp.concatenate([hd['ne']['b'] for hd in heads])[None, :]
        a1 = jnp.concatenate([hd['align']['w'][:H] for hd in heads], axis=1)
        a2 = jnp.zeros((2 * H, 2), jnp.float32)
        a2 = a2.at[:H, 0].set(heads[0]['align']['w'][H:, 0])
        a2 = a2.at[H:, 1].set(heads[1]['align']['w'][H:, 0])
        ba = jnp.stack([heads[0]['align']['b'][0],
                        heads[1]['align']['b'][0]])[None, :]
        wat = jnp.stack([hd['attend']['w'] for hd in heads])
        bat = jnp.stack([hd['attend']['b'] for hd in heads])[:, None, :]
        wg = jnp.stack([jnp.stack([hd['gru']['wz'], hd['gru']['wr'],
                                   hd['gru']['wn']]) for hd in heads])
        ug = jnp.stack([jnp.stack([hd['gru']['uz'], hd['gru']['ur'],
                                   hd['gru']['un']]) for hd in heads])
        bg = jnp.stack([jnp.stack([hd['gru']['bz'], hd['gru']['br'],
                                   hd['gru']['bn']]) for hd in heads])[:, :, None, :]
        return dict(w1=w1, w2=w2, bne=bne, a1=a1, a2=a2, ba=ba, wat=wat,
                    bat=bat, wg=wg, ug=ug, bg=bg)

    p1, p2 = prep(pc1), prep(pc2)

    x1, g1, pt1 = _tc_node_a(fn1, pc1['emb_fn']['w'], pc1['emb_fn']['b'][None, :],
                             p1['w1'], p1['a1'])
    x2, g2, pt2 = _tc_node_a(fn2, pc2['emb_fn']['w'], pc2['emb_fn']['b'][None, :],
                             p2['w1'], p2['a1'])

    gcat = jnp.concatenate([g1, g2])
    ptcat = jnp.concatenate([pt1, pt2])

    def padi(a, n, v):
        return jnp.pad(a, (0, n - a.shape[0]), constant_values=v)

    src_g = jnp.concatenate([padi(ei1[0], epc, 0), padi(ei2[0] + NF, epc, NF)])
    dst_g = jnp.concatenate([padi(ei1[1], epc, 0), padi(ei2[1] + NF, epc, NF)])
    dst_s = jnp.concatenate([padi(ei1[1], epc, NF), padi(ei2[1], epc, NF)])

    gs = _sc_gather(gcat, src_g)
    pd = _sc_gather(ptcat, dst_g, tc_tiling=False)

    def pade(a, n):
        return jnp.pad(a, ((0, n - a.shape[0]), (0, 0)))

    fecat = jnp.concatenate([pade(fe1, epc), pade(fe2, epc)])
    we = jnp.stack([pc1['emb_fe']['w'], pc2['emb_fe']['w']])
    be = jnp.stack([pc1['emb_fe']['b'][None, :], pc2['emb_fe']['b'][None, :]])
    w2 = jnp.stack([p1['w2'], p2['w2']])
    bne = jnp.stack([p1['bne'], p2['bne']])
    a2 = jnp.stack([p1['a2'], p2['a2']])
    ba = jnp.stack([p1['ba'], p2['ba']])
    s, mx = _tc_pass_a(gs, fecat, pd, we, be, w2, bne, a2, ba)
    pay = _tc_pass_b(gs, fecat, s, mx, we, be, w2, bne)
    if _SC_SCATTER_ENABLED:
        scat = _sc_scatter(pay, dst_s, nseg)
    else:
        seg = jnp.where(jnp.arange(ep) < epc, dst_s, dst_s + nseg)
        sc2 = jax.vmap(lambda p: jax.ops.segment_sum(p, seg, num_segments=2 * nseg))(pay)
        scat = sc2.reshape(5, 2, nseg, 32)
    scat = scat[:, :, :NF, :]

    xs = jnp.stack([x1, x2])
    wat = jnp.stack([p1['wat'], p2['wat']])
    bat = jnp.stack([p1['bat'], p2['bat']])
    wg = jnp.stack([p1['wg'], p2['wg']])
    ug = jnp.stack([p1['ug'], p2['ug']])
    bg = jnp.stack([p1['bg'], p2['bg']])
    return _tc_node_b(scat, xs, wat, bat, wg, ug, bg, NF)


def _gru(p, x, h):
    z = jax.nn.sigmoid(x @ p['wz'] + h @ p['uz'] + p['bz'])
    r = jax.nn.sigmoid(x @ p['wr'] + h @ p['ur'] + p['br'])
    n = jnp.tanh(x @ p['wn'] + r * (h @ p['un']) + p['bn'])
    return (1.0 - z) * n + z * h


def _atom2(heads, xs, ei, ef_emb, n):
    src, dst = ei[0], ei[1]
    W2 = jnp.concatenate([hd['ne']['w'][H:] for hd in heads], axis=1)
    bne = jnp.concatenate([hd['ne']['b'] for hd in heads])
    A2blk = jnp.zeros((2 * H, 2), jnp.float32)
    A2blk = A2blk.at[:H, 0].set(heads[0]['align']['w'][H:, 0])
    A2blk = A2blk.at[H:, 1].set(heads[1]['align']['w'][H:, 0])
    ba = jnp.stack([heads[0]['align']['b'][0], heads[1]['align']['b'][0]])
    G = jnp.concatenate([xs[h] @ heads[h]['ne']['w'][:H] for h in range(2)], axis=1)
    P = jnp.concatenate([xs[h] @ heads[h]['align']['w'][:H] for h in range(2)], axis=1)
    em = ef_emb @ W2 + bne
    m = jax.nn.leaky_relu(G[src] + em)
    s = jax.nn.leaky_relu(P[dst] + m @ A2blk + ba)
    e = jnp.exp(s - jnp.max(s, axis=0))
    num = jax.ops.segment_sum(m * jnp.repeat(e, H, axis=1), dst, num_segments=n)
    den = jax.ops.segment_sum(e, dst, num_segments=n)
    outs = []
    for h, hd in enumerate(heads):
        c = num[:, h * H:(h + 1) * H] / (den[:, h:h + 1] + 1e-9)
        c = jax.nn.elu(c @ hd['attend']['w'] + hd['attend']['b'])
        outs.append(_gru(hd['gru'], c, xs[h]))
    return outs


def _mol(p, h, seg, n, steps=2):
    sg = jax.ops.segment_sum(h, seg, num_segments=n)
    wa1 = p['align']['w'][:H]; wa2 = p['align']['w'][H:]; ba = p['align']['b'][0]
    r = (h @ wa2)[:, 0]
    for _ in range(steps):
        q = (sg @ wa1)[:, 0]
        s = jax.nn.leaky_relu(q[seg] + r + ba)
        e = jnp.exp(s - jnp.max(s))
        num = jax.ops.segment_sum(h * e[:, None], seg, num_segments=n)
        den = jax.ops.segment_sum(e, seg, num_segments=n)
        c = num / (den[:, None] + 1e-9)
        c = jax.nn.elu(c @ p['attend']['w'] + p['attend']['b'])
        sg = _gru(p['gru'], c, sg)
    return sg


def _mol4_frag(mols, hh, fb1, fb2, steps=2):
    rpc = _ceil_to(2 * NF, 2048)
    e4 = 2 * rpc
    nseg = _ceil_to(2 * NM + 1, 128)
    padr = rpc - 2 * NF

    def comp_rows(c, arrs):
        return jnp.concatenate([arrs[2 * c], arrs[2 * c + 1],
                                jnp.zeros((padr,) + arrs[0].shape[1:],
                                          arrs[0].dtype)])

    hlist = [hh[0, 0], hh[0, 1], hh[1, 0], hh[1, 1]]
    h4 = jnp.concatenate([comp_rows(0, hlist), comp_rows(1, hlist)])

    def seg_comp(fb, off):
        return jnp.concatenate([fb + off, fb + off + NM,
                                jnp.full((padr,), 2 * NM, jnp.int32) + off])

    seg_scat = jnp.concatenate([seg_comp(fb1, 0), seg_comp(fb2, 0)])
    seg_gidx = jnp.concatenate([
        jnp.concatenate([fb1, fb1 + NM, jnp.zeros((padr,), jnp.int32)]),
        jnp.concatenate([fb2 + 2 * NM, fb2 + 3 * NM,
                         jnp.zeros((padr,), jnp.int32)])])

    wa1s = jnp.stack([p['align']['w'][:H, 0] for p in mols])
    wa2s = jnp.stack([p['align']['w'][H:, 0] for p in mols])
    bas = [p['align']['b'][0] for p in mols]
    rl = [hlist[i] @ wa2s[i] + bas[i] for i in range(4)]
    r4 = jnp.concatenate([comp_rows(0, rl), comp_rows(1, rl)])

    sg = _sc_scatter(h4[None], seg_scat, nseg)[0]
    sg4 = jnp.stack([sg[0, :NM], sg[0, NM:2 * NM],
                     sg[1, :NM], sg[1, NM:2 * NM]])

    wats = jnp.stack([p['attend']['w'] for p in mols])
    bats = jnp.stack([p['attend']['b'] for p in mols])
    gw = {k: jnp.stack([p['gru'][k] for p in mols])
          for k in ('wz', 'uz', 'bz', 'wr', 'ur', 'br', 'wn', 'un', 'bn')}

    for _ in range(steps):
        q4 = jnp.einsum('pnd,pd->pn', sg4, wa1s)
        qtab = jnp.pad(q4.reshape(4 * NM, 1), ((0, 0), (0, 15)))
        qg = _sc_gather(qtab, seg_gidx, tc_tiling=False)[:, 0]
        s = jax.nn.leaky_relu(qg + r4)
        ms = [jnp.max(lax.dynamic_slice_in_dim(s, (i // 2) * rpc + (i % 2) * NF,
                                               NF))
              for i in range(4)]
        mrow = jnp.concatenate([
            jnp.full((NF,), ms[0]), jnp.full((NF,), ms[1]),
            jnp.full((padr,), ms[1]),
            jnp.full((NF,), ms[2]), jnp.full((NF,), ms[3]),
            jnp.full((padr,), ms[3])])
        e = jnp.exp(s - mrow)
        pay = jnp.concatenate([h4 * e[:, None],
                               jnp.pad(e[:, None], ((0, 0), (0, 15)))], axis=1)
        acc = _sc_scatter(pay[None], seg_scat, nseg)[0]
        num4 = jnp.stack([acc[0, :NM, :H], acc[0, NM:2 * NM, :H],
                          acc[1, :NM, :H], acc[1, NM:2 * NM, :H]])
        den4 = jnp.stack([acc[0, :NM, H], acc[0, NM:2 * NM, H],
                          acc[1, :NM, H], acc[1, NM:2 * NM, H]])[..., None]
        c4 = num4 / (den4 + 1e-9)
        c4 = jax.nn.elu(jnp.einsum('pnd,pde->pne', c4, wats) + bats[:, None, :])
        z = jax.nn.sigmoid(jnp.einsum('pnd,pde->pne', c4, gw['wz'])
                           + jnp.einsum('pnd,pde->pne', sg4, gw['uz'])
                           + gw['bz'][:, None, :])
        rr = jax.nn.sigmoid(jnp.einsum('pnd,pde->pne', c4, gw['wr'])
                            + jnp.einsum('pnd,pde->pne', sg4, gw['ur'])
                            + gw['br'][:, None, :])
        nn = jnp.tanh(jnp.einsum('pnd,pde->pne', c4, gw['wn'])
                      + rr * jnp.einsum('pnd,pde->pne', sg4, gw['un'])
                      + gw['bn'][:, None, :])
        sg4 = (1.0 - z) * nn + z * sg4
    return sg4


def _component_tail(pc, outs, mn, me, mei, mb):
    gm = jax.nn.relu(jnp.concatenate(outs, axis=-1) @ pc['frag_attend']['w']
                     + pc['frag_attend']['b'])
    mx = jax.nn.leaky_relu(mn @ pc['emb_mn']['w'] + pc['emb_mn']['b'])
    mee = jax.nn.leaky_relu(me @ pc['emb_me']['w'] + pc['emb_me']['b'])
    mx = jnp.concatenate([gm, mx], axis=-1)
    ys = [mx @ hd['proj']['w'] + hd['proj']['b'] for hd in pc['jheads']]
    hhj = _atom2([hd['atom'] for hd in pc['jheads']], ys, mei, mee, NM)
    jo = [_mol(pc['jheads'][i]['mol'], hhj[i], mb, B) for i in range(NH)]
    return jax.nn.relu(jnp.concatenate(jo, axis=-1) @ pc['motif_attend']['w']
                       + pc['motif_attend']['b'])


def _final_mlp_kernel(g1_ref, g2_ref, t_ref, w11, b11, w12, b12, w13, b13,
                      w3, b3, w4, b4, out_ref):
    h1 = jnp.maximum(g1_ref[...] @ w11[...] + b11[...], 0.0)
    h2 = jnp.maximum(g2_ref[...] @ w12[...] + b12[...], 0.0)
    h3 = jnp.maximum(t_ref[...] @ w13[...] + b13[...], 0.0)
    z = jnp.concatenate([h1, h2, h3], axis=-1)
    z = jnp.maximum(z @ w3[...] + b3[...], 0.0)
    out_ref[...] = z @ w4[...] + b4[...]


def kernel(frag_node_comp1, frag_edge_comp1, frag_edge_index_comp1, frag_batch_comp1,
           motif_node_comp1, motif_edge_comp1, motif_edge_index_comp1, motif_batch_comp1,
           frag_node_comp2, frag_edge_comp2, frag_edge_index_comp2, frag_batch_comp2,
           motif_node_comp2, motif_edge_comp2, motif_edge_index_comp2, motif_batch_comp2,
           Tb_comp1, Tc_comp1, Tb_comp2, Tc_comp2, params):
    pc1, pc2 = params['c1'], params['c2']
    hh = _frag_stage(pc1, pc2, frag_node_comp1, frag_edge_comp1,
                     frag_edge_index_comp1, frag_node_comp2, frag_edge_comp2,
                     frag_edge_index_comp2)
    mols = [pc1['fheads'][0]['mol'], pc1['fheads'][1]['mol'],
            pc2['fheads'][0]['mol'], pc2['fheads'][1]['mol']]
    sg4 = _mol4_frag(mols, hh, frag_batch_comp1, frag_batch_comp2)
    g1 = _component_tail(pc1, [sg4[0], sg4[1]],
                         motif_node_comp1, motif_edge_comp1,
                         motif_edge_index_comp1, motif_batch_comp1)
    g2 = _component_tail(pc2, [sg4[2], sg4[3]],
                         motif_node_comp2, motif_edge_comp2,
                         motif_edge_index_comp2, motif_batch_comp2)
    t = jnp.concatenate([Tb_comp1, Tc_comp1, Tb_comp2, Tc_comp2], axis=-1)
    p = params
    return pl.pallas_call(
        _final_mlp_kernel,
        out_shape=jax.ShapeDtypeStruct((B, 1), jnp.float32),
    )(g1, g2, t,
      p['p11']['w'], p['p11']['b'], p['p12']['w'], p['p12']['b'],
      p['p13']['w'], p['p13']['b'], p['p3']['w'], p['p3']['b'],
      p['p4']['w'], p['p4']['b'])
